# baseline (speedup 1.0000x reference)
import jax
import jax.numpy as jnp
from jax.experimental import pallas as pl

BATCH = 16384
NUM_FEATURES = 16384
NUM_CLASSES = 2

BM = 128


def _linear_kernel(x_ref, w_ref, b_ref, o_ref):
    x = x_ref[...]
    w = w_ref[...]
    acc = jax.lax.dot_general(
        x, w, (((1,), (1,)), ((), ())), preferred_element_type=jnp.float32
    )
    o_ref[...] = acc + b_ref[...]


def kernel(batch, W, b):
    b2 = b.reshape(1, NUM_CLASSES)
    grid = (BATCH // BM,)
    out = pl.pallas_call(
        _linear_kernel,
        grid=grid,
        in_specs=[
            pl.BlockSpec((BM, NUM_FEATURES), lambda i: (i, 0)),
            pl.BlockSpec((NUM_CLASSES, NUM_FEATURES), lambda i: (0, 0)),
            pl.BlockSpec((1, NUM_CLASSES), lambda i: (0, 0)),
        ],
        out_specs=pl.BlockSpec((BM, NUM_CLASSES), lambda i: (i, 0)),
        out_shape=jax.ShapeDtypeStruct((BATCH, NUM_CLASSES), jnp.float32),
    )(batch, W, b2)
    return out
